# baseline (speedup 1.0000x reference)
import functools

import jax
import jax.numpy as jnp
from jax import lax
from jax.experimental import pallas as pl
from jax.experimental.pallas import tpu as pltpu
from jax.experimental.pallas import tpu_sc as plsc

B = 8
Q = 300
D = 256
H = 8
HD = 32
P = 16
LVL_SIZE = (80.0, 40.0, 20.0, 10.0)
LVL_START = (0, 6400, 8000, 8400)
S = 8500
OFF_SCALE = 0.5

NC, NS = 2, 16
NW = NC * NS
WPB = NW // B
QPW = Q // WPB
QC = 2
NCH = -(-QPW // QC)


def _prep_body(hs_ref, rp_ref, wx_ref, wy_ref, wa_ref, bx_ref, by_ref,
               ba_ref, nps_ref, attn_ref, idx_ref, wt_ref):
    hs = hs_ref[...]
    hp = lax.Precision.HIGHEST
    offx = jnp.dot(hs, wx_ref[...], precision=hp) + bx_ref[...]
    offy = jnp.dot(hs, wy_ref[...], precision=hp) + by_ref[...]
    logits = jnp.dot(hs, wa_ref[...], precision=hp) + ba_ref[...]

    parts = []
    for g in range(H):
        xg = logits[:, g * P:(g + 1) * P]
        m = jnp.max(xg, axis=1, keepdims=True)
        e = jnp.exp(xg - m)
        parts.append(e / jnp.sum(e, axis=1, keepdims=True))
    attn = jnp.concatenate(parts, axis=1)
    attn_ref[...] = attn

    lane = lax.broadcasted_iota(jnp.int32, (1, 128), 1)
    h_idx = lane // P
    p_idx = lane % P
    lvl = p_idx // 4
    wl_f = jnp.where(lvl == 0, LVL_SIZE[0],
                     jnp.where(lvl == 1, LVL_SIZE[1],
                               jnp.where(lvl == 2, LVL_SIZE[2], LVL_SIZE[3])))
    start = jnp.where(lvl == 0, LVL_START[0],
                      jnp.where(lvl == 1, LVL_START[1],
                                jnp.where(lvl == 2, LVL_START[2], LVL_START[3])))
    wl_i = wl_f.astype(jnp.int32)

    brow = lax.broadcasted_iota(jnp.int32, (B * Q, 1), 0) // Q
    rx = rp_ref[:, 0:1]
    ry = rp_ref[:, 1:2]
    rw = rp_ref[:, 2:3]
    rh = rp_ref[:, 3:4]
    nps = nps_ref[...] * OFF_SCALE

    locx = rx + offx * nps * rw
    locy = ry + offy * nps * rh
    xg_ = locx * wl_f - 0.5
    yg_ = locy * wl_f - 0.5
    x0 = jnp.floor(xg_)
    fx = xg_ - x0
    y0 = jnp.floor(yg_)
    fy = yg_ - y0
    wmax = wl_f - 1.0
    one = jnp.float32(1.0)
    vx0 = ((x0 >= 0.0) & (x0 <= wmax)).astype(jnp.float32)
    vx1 = ((x0 + 1.0 >= 0.0) & (x0 + 1.0 <= wmax)).astype(jnp.float32)
    vy0 = ((y0 >= 0.0) & (y0 <= wmax)).astype(jnp.float32)
    vy1 = ((y0 + 1.0 >= 0.0) & (y0 + 1.0 <= wmax)).astype(jnp.float32)
    cx0 = jnp.clip(x0, 0.0, wmax).astype(jnp.int32)
    cx1 = jnp.clip(x0 + 1.0, 0.0, wmax).astype(jnp.int32)
    cy0 = jnp.clip(y0, 0.0, wmax).astype(jnp.int32)
    cy1 = jnp.clip(y0 + 1.0, 0.0, wmax).astype(jnp.int32)

    base = brow * S + start

    def tap(cx, cy, twx, twy, vx, vy):
        rows = (base + cy * wl_i + cx) * H + h_idx
        w = attn * (twx * twy) * (vx * vy)
        return rows, w

    taps = [tap(cx0, cy0, one - fx, one - fy, vx0, vy0),
            tap(cx1, cy0, fx, one - fy, vx1, vy0),
            tap(cx0, cy1, one - fx, fy, vx0, vy1),
            tap(cx1, cy1, fx, fy, vx1, vy1)]
    for t, (rows, w) in enumerate(taps):
        idx_ref[t] = rows
        wt_ref[t] = w


def _prep_call(hs2, rp2, wx, wy, wa, bx, by, ba, npst):
    return pl.pallas_call(
        _prep_body,
        out_shape=[
            jax.ShapeDtypeStruct((B * Q, 128), jnp.float32),
            jax.ShapeDtypeStruct((4, B * Q, 128), jnp.int32),
            jax.ShapeDtypeStruct((4, B * Q, 128), jnp.float32),
        ],
    )(hs2, rp2, wx, wy, wa, bx, by, ba, npst)


def _sc_body(table_hbm, idx_hbm, w_hbm, out_hbm,
             idxv0, idxv1, idxv2, wv0, wv1, wv2,
             vbuf0, vbuf1, vbuf2, outv,
             sem_i0, sem_i1, sem_i2, sem_g0, sem_g1, sem_g2):
    wid = lax.axis_index("s") * NC + lax.axis_index("c")
    b = wid // WPB
    wq = wid % WPB
    qbase = b * Q + wq * QPW

    idxv = (idxv0, idxv1, idxv2)
    wv = (wv0, wv1, wv2)
    vbuf = (vbuf0, vbuf1, vbuf2)
    sem_i = (sem_i0, sem_i1, sem_i2)
    sem_g = (sem_g0, sem_g1, sem_g2)
    last = NCH - 1

    def q0_of(c):
        return jnp.minimum(c * QC, QPW - QC)

    def copy_descs(c, s):
        rowblk = qbase + q0_of(c)
        descs = []
        for t in range(4):
            descs.append(pltpu.make_async_copy(
                idx_hbm.at[t, pl.ds(rowblk, QC)], idxv[s].at[t], sem_i[s]))
            descs.append(pltpu.make_async_copy(
                w_hbm.at[t, pl.ds(rowblk, QC)], wv[s].at[t], sem_i[s]))
        return descs

    def start_copies(c, s):
        for d_ in copy_descs(c, s):
            d_.start()

    def wait_copies(c, s):
        for d_ in copy_descs(c, s):
            d_.wait()

    def gather_descs(vs, cs):
        return [pltpu.make_async_copy(
                    table_hbm.at[idxv[cs].at[j % 4, j // 4]],
                    vbuf[vs].at[pl.ds(j * 128, 128)],
                    sem_g[vs])
                for j in range(QC * 4)]

    def fire_gathers(vs, cs):
        for d_ in gather_descs(vs, cs):
            d_.start()

    def wait_gathers(vs, cs):
        for d_ in gather_descs(vs, cs):
            d_.wait()

    def compute(c, vs, cs):
        q0 = q0_of(c)
        wv_s = wv[cs]
        vbuf_s = vbuf[vs]

        def qh_body(qh, c2):
            q = qh // H
            h = qh % H
            hp0 = h * P
            acc0 = jnp.zeros((16,), jnp.float32)
            acc1 = jnp.zeros((16,), jnp.float32)
            for t in range(4):
                wrow = wv_s[t, q, pl.ds(hp0, P)]
                for p in range(P):
                    w = wrow[p]
                    r = q * 512 + t * 128 + hp0 + p
                    acc0 = acc0 + w * vbuf_s[r, pl.ds(0, 16)]
                    acc1 = acc1 + w * vbuf_s[r, pl.ds(16, 16)]
            lrow = (q0 + q) * H + h
            outv[lrow, pl.ds(0, 16)] = acc0
            outv[lrow, pl.ds(16, 16)] = acc1
            return c2

        lax.fori_loop(0, QC * H, qh_body, 0)

    for s in range(3):
        start_copies(s, s)
    wait_copies(0, 0)
    fire_gathers(0, 0)
    wait_copies(1, 1)
    fire_gathers(1, 1)

    def tri(K, carry):
        for j in range(3):
            c = 3 * K + j
            s2 = (j + 2) % 3

            @pl.when(c <= last)
            def _():
                wait_gathers(j, j)
                compute(c, j, j)

            @pl.when(c + 3 <= last)
            def _():
                start_copies(c + 3, j)

            @pl.when(c + 2 <= last)
            def _():
                wait_copies(c + 2, s2)
                fire_gathers(s2, s2)

        return carry

    lax.fori_loop(0, (NCH + 2) // 3, tri, 0)
    pltpu.sync_copy(outv, out_hbm.at[pl.ds(qbase * H, QPW * H)])


@functools.lru_cache(maxsize=1)
def _sc_combine():
    return pl.kernel(
        _sc_body,
        out_type=jax.ShapeDtypeStruct((B * Q * H, HD), jnp.float32),
        mesh=plsc.VectorSubcoreMesh(core_axis_name="c", subcore_axis_name="s",
                                    num_cores=NC, num_subcores=NS),
        scratch_types=(
            [pltpu.VMEM((4, QC, 128), jnp.int32)] * 3
            + [pltpu.VMEM((4, QC, 128), jnp.float32)] * 3
            + [pltpu.VMEM((QC * 4 * 128, HD), jnp.float32)] * 3
            + [pltpu.VMEM((QPW * H, HD), jnp.float32)]
            + [pltpu.SemaphoreType.DMA] * 6
        ),
        compiler_params=pltpu.CompilerParams(use_tc_tiling_on_sc=False,
                                             needs_layout_passes=False),
    )


def kernel(hidden_states, reference_points, encoder_hidden_states,
           spatial_shapes, W_off, b_off, W_attn, b_attn, num_points_scale):
    hs2 = hidden_states.reshape(B * Q, D)
    rp2 = reference_points.reshape(B * Q, 4)
    lanes = jnp.arange(D)[:, None]
    selx = (lanes == jnp.arange(128)[None, :] * 2).astype(W_off.dtype)
    sely = (lanes == jnp.arange(128)[None, :] * 2 + 1).astype(W_off.dtype)
    wx = W_off @ selx
    wy = W_off @ sely
    bx = (b_off @ selx).reshape(1, 128)
    by = (b_off @ sely).reshape(1, 128)
    ba = b_attn.reshape(1, 128)
    npst = jnp.tile(num_points_scale, H).reshape(1, 128)

    attn, idx, wt = _prep_call(hs2, rp2, wx, wy, W_attn, bx, by, ba, npst)

    table = encoder_hidden_states.reshape(B * S * H, HD)
    out = _sc_combine()(table, idx, wt)

    output = out.reshape(B, Q, D)
    attn_out = attn.reshape(B, Q, H, P)
    return output, attn_out


# speedup vs baseline: 1.0257x; 1.0257x over previous
import functools

import jax
import jax.numpy as jnp
from jax import lax
from jax.experimental import pallas as pl
from jax.experimental.pallas import tpu as pltpu
from jax.experimental.pallas import tpu_sc as plsc

B = 8
Q = 300
D = 256
H = 8
HD = 32
P = 16
LVL_SIZE = (80.0, 40.0, 20.0, 10.0)
LVL_START = (0, 6400, 8000, 8400)
S = 8500
OFF_SCALE = 0.5

NC, NS = 2, 16
NW = NC * NS
WPB = NW // B
QPW = Q // WPB
QC = 3
NCH = QPW // QC


def _prep_body(hs_ref, rp_ref, wx_ref, wy_ref, wa_ref, bx_ref, by_ref,
               ba_ref, nps_ref, attn_ref, idx_ref, wt_ref):
    hs = hs_ref[...]
    hp = lax.Precision.HIGHEST
    offx = jnp.dot(hs, wx_ref[...], precision=hp) + bx_ref[...]
    offy = jnp.dot(hs, wy_ref[...], precision=hp) + by_ref[...]
    logits = jnp.dot(hs, wa_ref[...], precision=hp) + ba_ref[...]

    parts = []
    for g in range(H):
        xg = logits[:, g * P:(g + 1) * P]
        m = jnp.max(xg, axis=1, keepdims=True)
        e = jnp.exp(xg - m)
        parts.append(e / jnp.sum(e, axis=1, keepdims=True))
    attn = jnp.concatenate(parts, axis=1)
    attn_ref[...] = attn

    lane = lax.broadcasted_iota(jnp.int32, (1, 128), 1)
    h_idx = lane // P
    p_idx = lane % P
    lvl = p_idx // 4
    wl_f = jnp.where(lvl == 0, LVL_SIZE[0],
                     jnp.where(lvl == 1, LVL_SIZE[1],
                               jnp.where(lvl == 2, LVL_SIZE[2], LVL_SIZE[3])))
    start = jnp.where(lvl == 0, LVL_START[0],
                      jnp.where(lvl == 1, LVL_START[1],
                                jnp.where(lvl == 2, LVL_START[2], LVL_START[3])))
    wl_i = wl_f.astype(jnp.int32)

    brow = lax.broadcasted_iota(jnp.int32, (B * Q, 1), 0) // Q
    rx = rp_ref[:, 0:1]
    ry = rp_ref[:, 1:2]
    rw = rp_ref[:, 2:3]
    rh = rp_ref[:, 3:4]
    nps = nps_ref[...] * OFF_SCALE

    locx = rx + offx * nps * rw
    locy = ry + offy * nps * rh
    xg_ = locx * wl_f - 0.5
    yg_ = locy * wl_f - 0.5
    x0 = jnp.floor(xg_)
    fx = xg_ - x0
    y0 = jnp.floor(yg_)
    fy = yg_ - y0
    wmax = wl_f - 1.0
    one = jnp.float32(1.0)
    vx0 = ((x0 >= 0.0) & (x0 <= wmax)).astype(jnp.float32)
    vx1 = ((x0 + 1.0 >= 0.0) & (x0 + 1.0 <= wmax)).astype(jnp.float32)
    vy0 = ((y0 >= 0.0) & (y0 <= wmax)).astype(jnp.float32)
    vy1 = ((y0 + 1.0 >= 0.0) & (y0 + 1.0 <= wmax)).astype(jnp.float32)
    cx0 = jnp.clip(x0, 0.0, wmax).astype(jnp.int32)
    cx1 = jnp.clip(x0 + 1.0, 0.0, wmax).astype(jnp.int32)
    cy0 = jnp.clip(y0, 0.0, wmax).astype(jnp.int32)
    cy1 = jnp.clip(y0 + 1.0, 0.0, wmax).astype(jnp.int32)

    base = brow * S + start

    def tap(cx, cy, twx, twy, vx, vy):
        rows = (base + cy * wl_i + cx) * H + h_idx
        w = attn * (twx * twy) * (vx * vy)
        return rows, w

    taps = [tap(cx0, cy0, one - fx, one - fy, vx0, vy0),
            tap(cx1, cy0, fx, one - fy, vx1, vy0),
            tap(cx0, cy1, one - fx, fy, vx0, vy1),
            tap(cx1, cy1, fx, fy, vx1, vy1)]
    for t, (rows, w) in enumerate(taps):
        idx_ref[t] = rows
        wt_ref[t] = w


def _prep_call(hs2, rp2, wx, wy, wa, bx, by, ba, npst):
    return pl.pallas_call(
        _prep_body,
        out_shape=[
            jax.ShapeDtypeStruct((B * Q, 128), jnp.float32),
            jax.ShapeDtypeStruct((4, B * Q, 128), jnp.int32),
            jax.ShapeDtypeStruct((4, B * Q, 128), jnp.float32),
        ],
    )(hs2, rp2, wx, wy, wa, bx, by, ba, npst)


def _sc_body(table_hbm, idx_hbm, w_hbm, out_hbm,
             idxv0, idxv1, idxv2, idxv3, wv0, wv1, wv2, wv3,
             vbuf0, vbuf1, outv,
             sem_i0, sem_i1, sem_i2, sem_i3, sem_g0, sem_g1):
    wid = lax.axis_index("s") * NC + lax.axis_index("c")
    b = wid // WPB
    wq = wid % WPB
    qbase = b * Q + wq * QPW

    idxv = (idxv0, idxv1, idxv2, idxv3)
    wv = (wv0, wv1, wv2, wv3)
    vbuf = (vbuf0, vbuf1)
    sem_i = (sem_i0, sem_i1, sem_i2, sem_i3)
    sem_g = (sem_g0, sem_g1)
    last = NCH - 1

    def q0_of(c):
        return jnp.minimum(c, last) * QC

    def copy_descs(c, s):
        rowblk = qbase + q0_of(c)
        descs = []
        for t in range(4):
            descs.append(pltpu.make_async_copy(
                idx_hbm.at[t, pl.ds(rowblk, QC)], idxv[s].at[t], sem_i[s]))
            descs.append(pltpu.make_async_copy(
                w_hbm.at[t, pl.ds(rowblk, QC)], wv[s].at[t], sem_i[s]))
        return descs

    def start_copies(c, s):
        for d_ in copy_descs(c, s):
            d_.start()

    def wait_copies(c, s):
        for d_ in copy_descs(c, s):
            d_.wait()

    def gather_descs(vs, cs):
        return [pltpu.make_async_copy(
                    table_hbm.at[idxv[cs].at[j % 4, j // 4]],
                    vbuf[vs].at[pl.ds(j * 128, 128)],
                    sem_g[vs])
                for j in range(QC * 4)]

    def fire_gathers(vs, cs):
        for d_ in gather_descs(vs, cs):
            d_.start()

    def wait_gathers(vs, cs):
        for d_ in gather_descs(vs, cs):
            d_.wait()

    def compute(c, vs, cs):
        q0 = q0_of(c)
        wv_s = wv[cs]
        vbuf_s = vbuf[vs]

        def qh_body(qh, c2):
            q = qh // H
            h = qh % H
            hp0 = h * P
            acc0 = jnp.zeros((16,), jnp.float32)
            acc1 = jnp.zeros((16,), jnp.float32)
            for t in range(4):
                wrow = wv_s[t, q, pl.ds(hp0, P)]
                for p in range(P):
                    w = wrow[p]
                    r = q * 512 + t * 128 + hp0 + p
                    acc0 = acc0 + w * vbuf_s[r, pl.ds(0, 16)]
                    acc1 = acc1 + w * vbuf_s[r, pl.ds(16, 16)]
            lrow = (q0 + q) * H + h
            outv[lrow, pl.ds(0, 16)] = acc0
            outv[lrow, pl.ds(16, 16)] = acc1
            return c2

        lax.fori_loop(0, QC * H, qh_body, 0)

    for s in range(4):
        start_copies(s, s)
    wait_copies(0, 0)
    fire_gathers(0, 0)
    wait_copies(1, 1)
    fire_gathers(1, 1)

    def quad(K, carry):
        for j in range(4):
            c = 4 * K + j
            vs = j % 2
            cs = j

            @pl.when(c <= last)
            def _():
                wait_gathers(vs, cs)
                compute(c, vs, cs)

            @pl.when(c + 4 <= last)
            def _():
                start_copies(c + 4, cs)

            @pl.when(c + 2 <= last)
            def _():
                wait_copies(c + 2, (j + 2) % 4)
                fire_gathers(vs, (j + 2) % 4)

        return carry

    lax.fori_loop(0, (NCH + 3) // 4, quad, 0)
    pltpu.sync_copy(outv, out_hbm.at[pl.ds(qbase * H, QPW * H)])


@functools.lru_cache(maxsize=1)
def _sc_combine():
    return pl.kernel(
        _sc_body,
        out_type=jax.ShapeDtypeStruct((B * Q * H, HD), jnp.float32),
        mesh=plsc.VectorSubcoreMesh(core_axis_name="c", subcore_axis_name="s",
                                    num_cores=NC, num_subcores=NS),
        scratch_types=(
            [pltpu.VMEM((4, QC, 128), jnp.int32)] * 4
            + [pltpu.VMEM((4, QC, 128), jnp.float32)] * 4
            + [pltpu.VMEM((QC * 4 * 128, HD), jnp.float32)] * 2
            + [pltpu.VMEM((QPW * H, HD), jnp.float32)]
            + [pltpu.SemaphoreType.DMA] * 6
        ),
        compiler_params=pltpu.CompilerParams(use_tc_tiling_on_sc=False,
                                             needs_layout_passes=False),
    )


def kernel(hidden_states, reference_points, encoder_hidden_states,
           spatial_shapes, W_off, b_off, W_attn, b_attn, num_points_scale):
    hs2 = hidden_states.reshape(B * Q, D)
    rp2 = reference_points.reshape(B * Q, 4)
    lanes = jnp.arange(D)[:, None]
    selx = (lanes == jnp.arange(128)[None, :] * 2).astype(W_off.dtype)
    sely = (lanes == jnp.arange(128)[None, :] * 2 + 1).astype(W_off.dtype)
    wx = W_off @ selx
    wy = W_off @ sely
    bx = (b_off @ selx).reshape(1, 128)
    by = (b_off @ sely).reshape(1, 128)
    ba = b_attn.reshape(1, 128)
    npst = jnp.tile(num_points_scale, H).reshape(1, 128)

    attn, idx, wt = _prep_call(hs2, rp2, wx, wy, W_attn, bx, by, ba, npst)

    table = encoder_hidden_states.reshape(B * S * H, HD)
    out = _sc_combine()(table, idx, wt)

    output = out.reshape(B, Q, D)
    attn_out = attn.reshape(B, Q, H, P)
    return output, attn_out
